# baseline (speedup 1.0000x reference)
import functools
import math

import jax
import jax.numpy as jnp
import numpy as np
from jax import lax
from jax.experimental import pallas as pl
from jax.experimental.pallas import tpu as pltpu
from jax.experimental.pallas import tpu_sc as plsc

_MAX_LEN = 200
_HID = 64
_BATCH = 4096
_SCALE = math.sqrt(_HID)

_NW = 32
_B = _BATCH * _MAX_LEN
_BPW = _B // _NW
_SEQ_PER_CHUNK = 2
_CHUNK = _SEQ_PER_CHUNK * _MAX_LEN
_NCHUNK = _BPW // _CHUNK
_SUB = 80
_NSUB = _CHUNK // _SUB
_LANES = 16
_VPR = _HID // _LANES


def _pos_tiled_np():
    para = np.arange(_MAX_LEN, dtype=np.float32).reshape(-1, 1) / np.power(
        10000.0, np.arange(0, _HID, 2, dtype=np.float32) / _HID)
    pos = np.zeros((_MAX_LEN, _HID), dtype=np.float32)
    pos[:, 0::2] = np.sin(para)
    pos[:, 1::2] = np.cos(para)
    return np.tile(pos, (_SEQ_PER_CHUNK, 1))


_POS_TILED = _pos_tiled_np()


def _sc_body(table_hbm, idx_hbm, pos_hbm, out_hbm, idx_v, rows_v, pos_v, sem):
    wid = lax.axis_index("s") * 2 + lax.axis_index("c")
    base = wid * _BPW
    pltpu.sync_copy(pos_hbm, pos_v)

    def chunk_body(c, carry):
        row0 = base + c * _CHUNK
        pltpu.sync_copy(idx_hbm.at[pl.ds(row0, _CHUNK)], idx_v)
        copies = []
        for j in range(_NSUB):
            sl = pl.ds(j * _SUB, _SUB)
            copies.append(
                pltpu.async_copy(table_hbm.at[idx_v.at[sl]], rows_v.at[sl], sem))
        for cp in copies:
            cp.wait()

        def row_body(r, rc):
            for k in range(_VPR):
                sl = pl.ds(k * _LANES, _LANES)
                rows_v[r, sl] = rows_v[r, sl] * _SCALE + pos_v[r, sl]
            return rc

        lax.fori_loop(0, _CHUNK, row_body, 0)
        pltpu.sync_copy(rows_v, out_hbm.at[pl.ds(row0, _CHUNK)])
        return carry

    lax.fori_loop(0, _NCHUNK, chunk_body, 0)


@jax.jit
def _run(table, idx, pos):
    mesh = plsc.VectorSubcoreMesh(core_axis_name="c", subcore_axis_name="s")
    k = functools.partial(
        pl.kernel,
        mesh=mesh,
        out_type=jax.ShapeDtypeStruct((_B, _HID), jnp.float32),
        scratch_types=[
            pltpu.VMEM((_CHUNK,), jnp.int32),
            pltpu.VMEM((_CHUNK, _HID), jnp.float32),
            pltpu.VMEM((_CHUNK, _HID), jnp.float32),
            pltpu.SemaphoreType.DMA,
        ],
        compiler_params=pltpu.CompilerParams(use_tc_tiling_on_sc=False),
    )(_sc_body)
    return k(table, idx, pos)


def kernel(x, table):
    idx = x.reshape(-1).astype(jnp.int32)
    pos = jnp.asarray(_POS_TILED)
    out = _run(table, idx, pos)
    return out.reshape(_BATCH, _MAX_LEN, _HID)


# speedup vs baseline: 1.0811x; 1.0811x over previous
import functools
import math

import jax
import jax.numpy as jnp
import numpy as np
from jax import lax
from jax.experimental import pallas as pl
from jax.experimental.pallas import tpu as pltpu
from jax.experimental.pallas import tpu_sc as plsc

_MAX_LEN = 200
_HID = 64
_BATCH = 4096
_SCALE = math.sqrt(_HID)

_NW = 32
_B = _BATCH * _MAX_LEN
_BPW = _B // _NW
_SEQ_PER_CHUNK = 2
_CHUNK = _SEQ_PER_CHUNK * _MAX_LEN
_NCHUNK = _BPW // _CHUNK
_SUB = 80
_NSUB = _CHUNK // _SUB
_LANES = 16
_VPR = _HID // _LANES


def _pos_tiled_np():
    para = np.arange(_MAX_LEN, dtype=np.float32).reshape(-1, 1) / np.power(
        10000.0, np.arange(0, _HID, 2, dtype=np.float32) / _HID)
    pos = np.zeros((_MAX_LEN, _HID), dtype=np.float32)
    pos[:, 0::2] = np.sin(para)
    pos[:, 1::2] = np.cos(para)
    return np.tile(pos, (_SEQ_PER_CHUNK, 1))


_POS_TILED = _pos_tiled_np()


def _sc_body(table_hbm, idx_hbm, pos_hbm, out_hbm,
             idx_v, rows_v, pos_v, sg0, sg1, so0, so1):
    wid = lax.axis_index("s") * 2 + lax.axis_index("c")
    base = wid * _BPW
    sg = (sg0, sg1)
    so = (so0, so1)
    pltpu.sync_copy(pos_hbm, pos_v)

    def fire_gathers(c, b):
        pltpu.sync_copy(idx_hbm.at[pl.ds(base + c * _CHUNK, _CHUNK)],
                        idx_v.at[b])
        for j in range(_NSUB):
            sl = pl.ds(j * _SUB, _SUB)
            pltpu.async_copy(table_hbm.at[idx_v.at[b].at[sl]],
                             rows_v.at[b].at[sl], sg[b])

    def drain_gathers(b):
        for j in range(_NSUB):
            sl = pl.ds(j * _SUB, _SUB)
            pltpu.make_async_copy(table_hbm.at[idx_v.at[b].at[sl]],
                                  rows_v.at[b].at[sl], sg[b]).wait()

    def drain_out(c, b):
        pltpu.make_async_copy(rows_v.at[b],
                              out_hbm.at[pl.ds(base + c * _CHUNK, _CHUNK)],
                              so[b]).wait()

    fire_gathers(0, 0)

    def pair_body(p, carry):
        c0 = p * 2
        for b in range(2):
            c = c0 + b
            nb = 1 - b

            @pl.when(c + 1 < _NCHUNK)
            def _prefetch():
                @pl.when(c >= 1)
                def _():
                    drain_out(c - 1, nb)
                fire_gathers(c + 1, nb)

            drain_gathers(b)

            def row_body(r, rc):
                for k in range(_VPR):
                    sl = pl.ds(k * _LANES, _LANES)
                    rows_v[b, r, sl] = rows_v[b, r, sl] * _SCALE + pos_v[r, sl]
                return rc

            lax.fori_loop(0, _CHUNK, row_body, 0)
            pltpu.async_copy(rows_v.at[b],
                             out_hbm.at[pl.ds(base + c * _CHUNK, _CHUNK)],
                             so[b])
        return carry

    lax.fori_loop(0, _NCHUNK // 2, pair_body, 0)
    drain_out(_NCHUNK - 2, (_NCHUNK - 2) % 2)
    drain_out(_NCHUNK - 1, (_NCHUNK - 1) % 2)


@jax.jit
def _run(table, idx, pos):
    mesh = plsc.VectorSubcoreMesh(core_axis_name="c", subcore_axis_name="s")
    k = functools.partial(
        pl.kernel,
        mesh=mesh,
        out_type=jax.ShapeDtypeStruct((_B, _HID), jnp.float32),
        scratch_types=[
            pltpu.VMEM((2, _CHUNK), jnp.int32),
            pltpu.VMEM((2, _CHUNK, _HID), jnp.float32),
            pltpu.VMEM((_CHUNK, _HID), jnp.float32),
            pltpu.SemaphoreType.DMA,
            pltpu.SemaphoreType.DMA,
            pltpu.SemaphoreType.DMA,
            pltpu.SemaphoreType.DMA,
        ],
        compiler_params=pltpu.CompilerParams(use_tc_tiling_on_sc=False),
    )(_sc_body)
    return k(table, idx, pos)


def kernel(x, table):
    idx = x.reshape(-1).astype(jnp.int32)
    pos = jnp.asarray(_POS_TILED)
    out = _run(table, idx, pos)
    return out.reshape(_BATCH, _MAX_LEN, _HID)
